# baseline (speedup 1.0000x reference)
import functools

import jax
import jax.numpy as jnp
from jax import lax
from jax.experimental import pallas as pl
from jax.experimental.pallas import tpu as pltpu
from jax.experimental.pallas import tpu_sc as plsc

_N = 10000
_E = 320000
_D = 128
_R = 8

_NC = 2
_NS = 16
_NT = _NC * _NS
_CH = 128
_NCHUNK = 80
_SECN = 16
_NSEC = _NCHUNK // _SECN
_EPT = _CH * _NCHUNK
_EPAD = _NT * _EPT
_NPAD = 10240
_RPT = _NPAD // _NS

_BN = 400
_NB = _N // _BN


def _xw_body(x_ref, w_ref, o_ref):
    o_ref[0] = jnp.dot(x_ref[...], w_ref[0], preferred_element_type=jnp.float32)


def _xw(x, W):
    return pl.pallas_call(
        _xw_body,
        grid=(_NB, _R),
        in_specs=[
            pl.BlockSpec((_BN, _D), lambda i, r: (i, 0)),
            pl.BlockSpec((1, _D, _D), lambda i, r: (r, 0, 0)),
        ],
        out_specs=pl.BlockSpec((1, _BN, _D), lambda i, r: (r, i, 0)),
        out_shape=jax.ShapeDtypeStruct((_R, _N, _D), jnp.float32),
    )(x, W)


def _selfp_body(x_ref, w_ref, o_ref):
    o_ref[...] = jnp.dot(x_ref[...], w_ref[...], preferred_element_type=jnp.float32)


def _selfp(x, Wself):
    return pl.pallas_call(
        _selfp_body,
        grid=(_NB,),
        in_specs=[
            pl.BlockSpec((_BN, _D), lambda i: (i, 0)),
            pl.BlockSpec((_D, _D), lambda i: (0, 0)),
        ],
        out_specs=pl.BlockSpec((_BN, _D), lambda i: (i, 0)),
        out_shape=jax.ShapeDtypeStruct((_N, _D), jnp.float32),
    )(x, Wself)


def _gidx_body(et_ref, src_ref, o_ref):
    o_ref[...] = et_ref[...] * _N + src_ref[...]


def _gidx(et2d, src2d):
    rows = et2d.shape[0]
    return pl.pallas_call(
        _gidx_body,
        grid=(2,),
        in_specs=[
            pl.BlockSpec((rows // 2, _CH), lambda i: (i, 0)),
            pl.BlockSpec((rows // 2, _CH), lambda i: (i, 0)),
        ],
        out_specs=pl.BlockSpec((rows // 2, _CH), lambda i: (i, 0)),
        out_shape=jax.ShapeDtypeStruct((rows, _CH), jnp.int32),
    )(et2d, src2d)


def _make_sc_agg():
    mesh = plsc.VectorSubcoreMesh(core_axis_name="c", subcore_axis_name="s")

    @functools.partial(
        pl.kernel,
        mesh=mesh,
        out_type=jax.ShapeDtypeStruct((_NC, _NS, _RPT, _D), jnp.float32),
        scratch_types=[
            [pltpu.VMEM((_SECN, _CH), jnp.int32)] * 2,
            [pltpu.VMEM((_SECN, _CH), jnp.int32)] * 2,
            pltpu.VMEM((2, _CH, _D), jnp.float32),
            pltpu.VMEM_SHARED((_NPAD, _D), jnp.float32),
            pltpu.SemaphoreType.DMA,
            pltpu.SemaphoreType.DMA,
            pltpu.SemaphoreType.DMA,
            [pltpu.SemaphoreType.DMA] * 2,
        ],
    )
    def sc_agg(xw_hbm, gidx_hbm, didx_hbm, zeros_hbm, out_hbm,
               gidx_v, didx_v, rows_v, agg_sh, sem0, sem1, semz, isems):
        c = lax.axis_index("c")
        s = lax.axis_index("s")
        row0 = s * _RPT
        sems = (sem0, sem1)

        def fetch_idx(k, ib):
            pltpu.async_copy(gidx_hbm.at[c, s, k], gidx_v[ib], isems[ib])
            pltpu.async_copy(didx_hbm.at[c, s, k], didx_v[ib], isems[ib])

        def wait_idx(k, ib):
            pltpu.make_async_copy(gidx_hbm.at[c, s, k], gidx_v[ib],
                                  isems[ib]).wait()
            pltpu.make_async_copy(didx_hbm.at[c, s, k], didx_v[ib],
                                  isems[ib]).wait()

        cz = pltpu.async_copy(zeros_hbm.at[pl.ds(row0, _RPT)],
                              agg_sh.at[pl.ds(row0, _RPT)], semz)
        fetch_idx(0, 0)
        wait_idx(0, 0)
        cz.wait()
        plsc.subcore_barrier()

        fetch_idx(1, 1)
        pltpu.async_copy(xw_hbm.at[gidx_v[0].at[0]], rows_v.at[0], sems[0])
        for g in range(_NCHUNK):
            ib = (g // _SECN) % 2
            b = g % 2
            if g % _SECN == 0 and 0 < g and g // _SECN + 1 < _NSEC:
                fetch_idx(g // _SECN + 1, 1 - ib)
            jn = g + 1
            if jn < _NCHUNK:
                nib = (jn // _SECN) % 2
                if jn % _SECN == 0:
                    wait_idx(jn // _SECN, nib)
                pltpu.async_copy(xw_hbm.at[gidx_v[nib].at[jn % _SECN]],
                                 rows_v.at[jn % 2], sems[jn % 2])
            pltpu.make_async_copy(xw_hbm.at[gidx_v[ib].at[g % _SECN]],
                                  rows_v.at[b], sems[b]).wait()
            pltpu.sync_copy(rows_v.at[b],
                            agg_sh.at[didx_v[ib].at[g % _SECN]], add=True)
        plsc.subcore_barrier()
        pltpu.sync_copy(agg_sh.at[pl.ds(row0, _RPT)], out_hbm.at[c, s])

    return sc_agg


_sc_agg = _make_sc_agg()


def _combine1_body(p_ref, sp_ref, b_ref, o_ref):
    o_ref[...] = jnp.maximum(
        p_ref[0] + p_ref[1] + sp_ref[...] + b_ref[...], 0.0)


def _combine1(p, sp, b):
    return pl.pallas_call(
        _combine1_body,
        grid=(_NB,),
        in_specs=[
            pl.BlockSpec((2, _BN, _D), lambda i: (0, i, 0)),
            pl.BlockSpec((_BN, _D), lambda i: (i, 0)),
            pl.BlockSpec((1, _D), lambda i: (0, 0)),
        ],
        out_specs=pl.BlockSpec((_BN, _D), lambda i: (i, 0)),
        out_shape=jax.ShapeDtypeStruct((_N, _D), jnp.float32),
    )(p, sp, b)


def _combine2_body(p_ref, sp_ref, b_ref, fcw_ref, fcb_ref, o_ref, acc_ref):
    i = pl.program_id(0)

    @pl.when(i == 0)
    def _():
        acc_ref[...] = jnp.zeros_like(acc_ref)

    h = jnp.maximum(p_ref[0] + p_ref[1] + sp_ref[...] + b_ref[...], 0.0)
    acc_ref[0:1] += jnp.sum(h, axis=0, keepdims=True)

    @pl.when(i == pl.num_programs(0) - 1)
    def _():
        hg = acc_ref[0:1] * (1.0 / _N)
        z = jnp.sum(hg * fcw_ref[...], keepdims=True) + fcb_ref[...]
        o_ref[...] = 1.0 / (1.0 + jnp.exp(-z))


def _combine2(p, sp, b, fcw_row, fcb):
    return pl.pallas_call(
        _combine2_body,
        grid=(_NB,),
        in_specs=[
            pl.BlockSpec((2, _BN, _D), lambda i: (0, i, 0)),
            pl.BlockSpec((_BN, _D), lambda i: (i, 0)),
            pl.BlockSpec((1, _D), lambda i: (0, 0)),
            pl.BlockSpec((1, _D), lambda i: (0, 0)),
            pl.BlockSpec((1, 1), lambda i: (0, 0)),
        ],
        out_specs=pl.BlockSpec((1, 1), lambda i: (0, 0)),
        out_shape=jax.ShapeDtypeStruct((1, 1), jnp.float32),
        scratch_shapes=[pltpu.VMEM((8, _D), jnp.float32)],
    )(p, sp, b, fcw_row, fcb)


def kernel(in_feat, edge_index, e_types, W1, Wself1, b1, W2, Wself2, b2,
           fc_w, fc_b):
    src = edge_index[0]
    dst = edge_index[1]
    pad = _EPAD - _E
    et_p = jnp.concatenate([e_types, jnp.zeros((pad,), jnp.int32)])
    src_p = jnp.concatenate([src, jnp.zeros((pad,), jnp.int32)])
    dst_p = jnp.concatenate([dst, jnp.full((pad,), _N, jnp.int32)])

    gidx = _gidx(et_p.reshape(-1, _CH), src_p.reshape(-1, _CH))
    gidx4 = gidx.reshape(_NC, _NS, _NSEC, _SECN, _CH)
    didx4 = dst_p.reshape(_NC, _NS, _NSEC, _SECN, _CH)
    zeros = jnp.zeros((_NPAD, _D), jnp.float32)

    def layer(x, W, Wself):
        xw = _xw(x, W)
        sp = _selfp(x, Wself)
        p = _sc_agg(xw.reshape(_R * _N, _D), gidx4, didx4, zeros)
        return p.reshape(_NC, _NPAD, _D), sp

    p1, sp1 = layer(in_feat, W1, Wself1)
    h1 = _combine1(p1, sp1, b1.reshape(1, _D))
    p2, sp2 = layer(h1, W2, Wself2)
    return _combine2(p2, sp2, b2.reshape(1, _D), fc_w.reshape(1, _D),
                     fc_b.reshape(1, 1))
